# Initial kernel scaffold; baseline (speedup 1.0000x reference)
import jax
import jax.numpy as jnp
from jax.experimental import pallas as pl


def kernel(x, MM, PM, params):
    raise NotImplementedError("write your pallas kernel here")


# speedup vs baseline: 9.1244x; 9.1244x over previous
import jax
import jax.numpy as jnp
from jax.experimental import pallas as pl


def _dot(a, b):
    return jnp.dot(a, b, preferred_element_type=jnp.float32)


def _fused_kernel(x_ref, mm_ref,
                  w11, b11, w12, b12, w13, b13, w14, b14,
                  w21, b21, w22, b22, w23, b23, w24, b24,
                  w31, b31, w32, b32, w33, b33, w34, b34,
                  w41, b41, w42, b42, w43, b43, w44, b44,
                  out_ref):
    xc = x_ref[0]
    init = xc
    mm = mm_ref[0]
    mmc = mm[:, None]
    mmr = mm[None, :]
    prev = (mmr == mmc - 1).astype(jnp.float32)
    nxt = (mmr == mmc + 1).astype(jnp.float32)

    h3 = jax.nn.relu(b31[...][None, :])
    h3 = jax.nn.relu(_dot(h3, w32[...]) + b32[...])
    h3 = jax.nn.relu(_dot(h3, w33[...]) + b33[...])
    a3 = jax.nn.relu(_dot(h3, w34[...]) + b34[...])

    c_const = _dot(a3, w41[16:24, :]) + _dot(init, w41[40:48, :]) + b41[...][None, :]

    for _ in range(3):
        in1 = _dot(prev, xc)
        in2 = _dot(nxt, xc)

        h = jax.nn.relu(_dot(in1, w11[...]) + b11[...])
        h = jax.nn.relu(_dot(h, w12[...]) + b12[...])
        h = jax.nn.relu(_dot(h, w13[...]) + b13[...])
        a1 = jax.nn.relu(_dot(h, w14[...]) + b14[...])

        h = jax.nn.relu(_dot(in2, w21[...]) + b21[...])
        h = jax.nn.relu(_dot(h, w22[...]) + b22[...])
        h = jax.nn.relu(_dot(h, w23[...]) + b23[...])
        a2 = jax.nn.relu(_dot(h, w24[...]) + b24[...])

        a4 = jax.nn.relu(jnp.sum(xc, axis=0, keepdims=True))

        h = (_dot(a1, w41[0:8, :]) + _dot(a2, w41[8:16, :])
             + _dot(a4, w41[24:32, :]) + _dot(xc, w41[32:40, :]) + c_const)
        h = jax.nn.relu(h)
        h = jax.nn.relu(_dot(h, w42[...]) + b42[...])
        h = jax.nn.relu(_dot(h, w43[...]) + b43[...])
        xc = _dot(h, w44[...]) + b44[...]

    out_ref[0] = xc


def kernel(x, MM, PM, params):
    J, N, d = x.shape
    flat = []
    for name in ("f1", "f2", "f3", "f4"):
        for W, b in params[name]:
            flat.append(W)
            flat.append(b)
    out = pl.pallas_call(
        _fused_kernel,
        out_shape=jax.ShapeDtypeStruct((J, N, d), jnp.float32),
    )(x, MM, *flat)
    return out
